# Initial kernel scaffold; baseline (speedup 1.0000x reference)
import jax
import jax.numpy as jnp
from jax.experimental import pallas as pl


def kernel(query_keys, query_start_nodes, writer_keys, writer_labels, writer_start_nodes, W_key, b_key, class_embed, role_embed, start_embed, ln_g, ln_b, home_hash, expert_ln_g, expert_ln_b, W1, b1, W2, b2, W_out, b_out, W_delay, b_delay, W_mag, b_mag):
    raise NotImplementedError("write your pallas kernel here")


# speedup vs baseline: 1.3458x; 1.3458x over previous
import functools

import jax
import jax.numpy as jnp
from jax import lax
from jax.experimental import pallas as pl
from jax.experimental.pallas import tpu as pltpu
from jax.experimental.pallas import tpu_sc as plsc

F32 = jnp.float32
BF16 = jnp.bfloat16
T = 3072
TW = 1024
D = 1024
KD = 128
FFD = 2048
E = 8
NCLS = 256
BLK = 256
NB = 32
PMAX = NB * BLK
NWRK = 32
TPW = T // NWRK
CH = 48
CPW = TW // NWRK


def _sc_gather_embed(nodes_all, labels, start_embed, class_embed):
    mesh = plsc.VectorSubcoreMesh(core_axis_name="c", subcore_axis_name="s")

    @functools.partial(
        pl.kernel, mesh=mesh,
        out_type=(jax.ShapeDtypeStruct((T, D), F32),
                  jax.ShapeDtypeStruct((TW, D), F32)),
        scratch_types=[pltpu.VMEM((CH,), jnp.int32),
                       pltpu.VMEM((CH, D), F32),
                       pltpu.VMEM((CPW,), jnp.int32),
                       pltpu.VMEM((CPW, D), F32),
                       pltpu.SemaphoreType.DMA],
    )
    def k(nodes_h, labels_h, start_h, class_h, ext_h, cls_h,
          idx_v, rows_v, lidx_v, crows_v, sem):
        wid = lax.axis_index("s") * 2 + lax.axis_index("c")
        base = wid * TPW
        for h in range(TPW // CH):
            off = base + h * CH
            pltpu.sync_copy(nodes_h.at[pl.ds(off, CH)], idx_v)
            pltpu.async_copy(start_h.at[idx_v], rows_v, sem).wait()
            pltpu.sync_copy(rows_v, ext_h.at[pl.ds(off, CH)])
        cb = wid * CPW
        pltpu.sync_copy(labels_h.at[pl.ds(cb, CPW)], lidx_v)
        pltpu.async_copy(class_h.at[lidx_v], crows_v, sem).wait()
        pltpu.sync_copy(crows_v, cls_h.at[pl.ds(cb, CPW)])

    return k(nodes_all, labels, start_embed, class_embed)


def _sc_scatter_rows(xnorm, dest1, dest2):
    mesh = plsc.VectorSubcoreMesh(core_axis_name="c", subcore_axis_name="s")

    @functools.partial(
        pl.kernel, mesh=mesh,
        out_type=jax.ShapeDtypeStruct((PMAX, D), F32),
        scratch_types=[pltpu.VMEM((CH,), jnp.int32),
                       pltpu.VMEM((CH, D), F32),
                       pltpu.SemaphoreType.DMA],
    )
    def k(xn_h, d1_h, d2_h, xs_h, idx_v, rows_v, sem):
        wid = lax.axis_index("s") * 2 + lax.axis_index("c")
        base = wid * TPW
        for h in range(TPW // CH):
            off = base + h * CH
            pltpu.sync_copy(xn_h.at[pl.ds(off, CH)], rows_v)
            pltpu.sync_copy(d1_h.at[pl.ds(off, CH)], idx_v)
            pltpu.async_copy(rows_v, xs_h.at[idx_v], sem).wait()
            pltpu.sync_copy(d2_h.at[pl.ds(off, CH)], idx_v)
            pltpu.async_copy(rows_v, xs_h.at[idx_v], sem).wait()

    return k(xnorm, dest1, dest2)


def _sc_gather_rows(y_sorted, dest1, dest2):
    mesh = plsc.VectorSubcoreMesh(core_axis_name="c", subcore_axis_name="s")

    @functools.partial(
        pl.kernel, mesh=mesh,
        out_type=(jax.ShapeDtypeStruct((T, D), F32),
                  jax.ShapeDtypeStruct((T, D), F32)),
        scratch_types=[pltpu.VMEM((CH,), jnp.int32),
                       pltpu.VMEM((CH, D), F32),
                       pltpu.SemaphoreType.DMA],
    )
    def k(ys_h, d1_h, d2_h, r1_h, r2_h, idx_v, rows_v, sem):
        wid = lax.axis_index("s") * 2 + lax.axis_index("c")
        base = wid * TPW
        for h in range(TPW // CH):
            off = base + h * CH
            pltpu.sync_copy(d1_h.at[pl.ds(off, CH)], idx_v)
            pltpu.async_copy(ys_h.at[idx_v], rows_v, sem).wait()
            pltpu.sync_copy(rows_v, r1_h.at[pl.ds(off, CH)])
            pltpu.sync_copy(d2_h.at[pl.ds(off, CH)], idx_v)
            pltpu.async_copy(ys_h.at[idx_v], rows_v, sem).wait()
            pltpu.sync_copy(rows_v, r2_h.at[pl.ds(off, CH)])

    return k(y_sorted, dest1, dest2)


def _encode_call(keys_all, ext, cls_rows, labels_col, W_key, b_key2,
                 role_embed, ln_g2, ln_b2, home_pad):
    nblk = T // BLK
    wblk = TW // BLK

    def body(keys_ref, ext_ref, cls_ref, lab_ref, wk_ref, bk_ref, role_ref,
             g_ref, b_ref, hh_ref, tok_ref, xn_ref, lg_ref, meta_ref):
        i = pl.program_id(0)
        k = keys_ref[...]
        learned = jnp.dot(k, wk_ref[...], precision=lax.Precision.HIGHEST,
                          preferred_element_type=F32)
        learned = learned + bk_ref[...] + ext_ref[...] + role_ref[0]
        wmask = jnp.where(i < wblk, 1.0, 0.0).astype(F32)
        learned = learned + wmask * cls_ref[...]
        mu = jnp.mean(learned, axis=-1, keepdims=True)
        var = jnp.mean((learned - mu) ** 2, axis=-1, keepdims=True)
        ln = (learned - mu) / jnp.sqrt(var + 1e-5) * g_ref[...] + b_ref[...]
        col = lax.broadcasted_iota(jnp.int32, (BLK, D), 1)
        lab = lab_ref[...]
        oh = jnp.where(col == lab + KD, 1.0, 0.0).astype(F32)
        kpad = jnp.concatenate([k, jnp.zeros((BLK, D - KD), F32)], axis=1)
        tok = ln + kpad + oh
        tok_ref[...] = tok
        mu2 = jnp.mean(tok, axis=-1, keepdims=True)
        var2 = jnp.mean((tok - mu2) ** 2, axis=-1, keepdims=True)
        xn_ref[...] = (tok - mu2) / jnp.sqrt(var2 + 1e-5)
        lg = jnp.dot(k, hh_ref[...], precision=lax.Precision.HIGHEST,
                     preferred_element_type=F32)
        lg_ref[...] = lg
        lane = lax.broadcasted_iota(jnp.int32, (BLK, 128), 1)
        lanef = lane.astype(F32)
        neg = jnp.float32(-1e30)
        masked = jnp.where(lane < E, lg, neg)
        m1 = jnp.max(masked, axis=-1, keepdims=True)
        i1 = jnp.min(jnp.where(masked == m1, lanef, jnp.float32(1e9)),
                     axis=-1, keepdims=True)
        masked2 = jnp.where(lanef == i1, neg, masked)
        m2 = jnp.max(masked2, axis=-1, keepdims=True)
        i2 = jnp.min(jnp.where(masked2 == m2, lanef, jnp.float32(1e9)),
                     axis=-1, keepdims=True)
        g1 = 1.0 / (1.0 + jnp.exp(m2 - m1))
        g2 = 1.0 - g1
        meta_ref[...] = jnp.where(
            lane == 0, g1,
            jnp.where(lane == 1, g2,
                      jnp.where(lane == 2, i1,
                                jnp.where(lane == 3, i2, 0.0))))

    return pl.pallas_call(
        body,
        grid=(nblk,),
        in_specs=[
            pl.BlockSpec((BLK, KD), lambda i: (i, 0)),
            pl.BlockSpec((BLK, D), lambda i: (i, 0)),
            pl.BlockSpec((BLK, D), lambda i: (jnp.minimum(i, wblk - 1), 0)),
            pl.BlockSpec((BLK, 1), lambda i: (i, 0)),
            pl.BlockSpec((KD, D), lambda i: (0, 0)),
            pl.BlockSpec((1, D), lambda i: (0, 0)),
            pl.BlockSpec((1, 1, D), lambda i: (jnp.where(i < wblk, 0, 1), 0, 0)),
            pl.BlockSpec((1, D), lambda i: (0, 0)),
            pl.BlockSpec((1, D), lambda i: (0, 0)),
            pl.BlockSpec((KD, 128), lambda i: (0, 0)),
        ],
        out_specs=[
            pl.BlockSpec((BLK, D), lambda i: (i, 0)),
            pl.BlockSpec((BLK, D), lambda i: (i, 0)),
            pl.BlockSpec((BLK, 128), lambda i: (i, 0)),
            pl.BlockSpec((BLK, 128), lambda i: (i, 0)),
        ],
        out_shape=[
            jax.ShapeDtypeStruct((T, D), F32),
            jax.ShapeDtypeStruct((T, D), F32),
            jax.ShapeDtypeStruct((T, 128), F32),
            jax.ShapeDtypeStruct((T, 128), F32),
        ],
    )(keys_all, ext, cls_rows, labels_col, W_key, b_key2, role_embed,
      ln_g2, ln_b2, home_pad)


def _route_call(meta):
    nchunk = T // BLK

    def body(meta_ref, d1_ref, d2_ref, be_ref, e1_scr, e2_scr, r1_scr, r2_scr):
        lane = lax.broadcasted_iota(jnp.int32, (BLK, 128), 1)
        lanef = lane.astype(F32)

        def ext_body(i, _):
            m = meta_ref[pl.ds(i * BLK, BLK), :]
            e1 = jnp.sum(jnp.where(lane == 2, m, 0.0), axis=1, keepdims=True)
            e2 = jnp.sum(jnp.where(lane == 3, m, 0.0), axis=1, keepdims=True)
            e1_scr[pl.ds(i * BLK, BLK), :] = e1
            e2_scr[pl.ds(i * BLK, BLK), :] = e2
            return 0

        lax.fori_loop(0, nchunk, ext_body, 0)

        row = lax.broadcasted_iota(jnp.int32, (BLK, BLK), 0)
        col = lax.broadcasted_iota(jnp.int32, (BLK, BLK), 1)
        tri = jnp.where(col < row, 1.0, 0.0).astype(F32)

        def rank_pass(e_scr, r_scr):
            def bdy(i, carry):
                e = e_scr[pl.ds(i * BLK, BLK), :]
                a = jnp.where((lanef == e) & (lane < E), 1.0, 0.0)
                ex = jnp.dot(tri, a, precision=lax.Precision.HIGHEST,
                             preferred_element_type=F32) + carry
                r_scr[pl.ds(i * BLK, BLK), :] = jnp.sum(
                    a * ex, axis=1, keepdims=True)
                return carry + jnp.sum(a, axis=0, keepdims=True)
            return bdy

        c1 = lax.fori_loop(0, nchunk, rank_pass(e1_scr, r1_scr),
                           jnp.zeros((1, 128), F32))
        counts = lax.fori_loop(0, nchunk, rank_pass(e2_scr, r2_scr), c1)

        pc = jnp.ceil(counts / BLK) * BLK
        r8 = lax.broadcasted_iota(jnp.int32, (128, 128), 0)
        c8 = lax.broadcasted_iota(jnp.int32, (128, 128), 1)
        upper = jnp.where(r8 < c8, 1.0, 0.0).astype(F32)
        offs = jnp.dot(pc, upper, precision=lax.Precision.HIGHEST,
                       preferred_element_type=F32)

        def dest_pass(e_scr, r_scr, d_ref):
            def bdy(i, _):
                e = e_scr[pl.ds(i * BLK, BLK), :]
                a = jnp.where((lanef == e) & (lane < E), 1.0, 0.0)
                oa = jnp.sum(a * offs, axis=1, keepdims=True)
                rank = r_scr[pl.ds(i * BLK, BLK), :]
                d_ref[pl.ds(i * BLK, BLK), :] = (oa + rank).astype(jnp.int32)
                return 0
            lax.fori_loop(0, nchunk, bdy, 0)

        dest_pass(e1_scr, r1_scr, d1_ref)
        dest_pass(e2_scr, r2_scr, d2_ref)

        nb = pc / BLK
        cumnb = jnp.dot(nb, upper, precision=lax.Precision.HIGHEST,
                        preferred_element_type=F32)
        tb = jnp.sum(nb)
        lane_b = lax.broadcasted_iota(jnp.int32, (NB, 128), 1)
        jrow = lax.broadcasted_iota(jnp.int32, (NB, 128), 0).astype(F32)
        ge = jnp.where((jrow >= cumnb) & (lane_b < E), 1.0, 0.0)
        be = jnp.sum(ge, axis=1, keepdims=True) - 1.0
        be = jnp.where(jrow[:, :1] < tb, be, -1.0)
        be_ref[...] = be.astype(jnp.int32)

    return pl.pallas_call(
        body,
        out_shape=[
            jax.ShapeDtypeStruct((T, 1), jnp.int32),
            jax.ShapeDtypeStruct((T, 1), jnp.int32),
            jax.ShapeDtypeStruct((NB, 1), jnp.int32),
        ],
        scratch_shapes=[
            pltpu.VMEM((T, 1), F32),
            pltpu.VMEM((T, 1), F32),
            pltpu.VMEM((T, 1), F32),
            pltpu.VMEM((T, 1), F32),
        ],
    )(meta)


def _ffn_call(be, x_sorted, eg, eb, W1b, b1, W2b, b2):
    def body(be_ref, x_ref, g_ref, bb_ref, w1_ref, b1_ref, w2_ref, b2_ref,
             y_ref):
        i = pl.program_id(0)

        @pl.when(be_ref[i] >= 0)
        def _():
            h = x_ref[...] * g_ref[0] + bb_ref[0]
            a = jnp.dot(h.astype(BF16), w1_ref[0],
                        preferred_element_type=F32) + b1_ref[0]
            act = jax.nn.gelu(a)
            y_ref[...] = jnp.dot(act.astype(BF16), w2_ref[0],
                                 preferred_element_type=F32) + b2_ref[0]

    def ei(i, b):
        return jnp.maximum(b[i], 0)

    grid_spec = pltpu.PrefetchScalarGridSpec(
        num_scalar_prefetch=1,
        grid=(NB,),
        in_specs=[
            pl.BlockSpec((BLK, D), lambda i, b: (i, 0)),
            pl.BlockSpec((1, 1, D), lambda i, b: (ei(i, b), 0, 0)),
            pl.BlockSpec((1, 1, D), lambda i, b: (ei(i, b), 0, 0)),
            pl.BlockSpec((1, D, FFD), lambda i, b: (ei(i, b), 0, 0)),
            pl.BlockSpec((1, 1, FFD), lambda i, b: (ei(i, b), 0, 0)),
            pl.BlockSpec((1, FFD, D), lambda i, b: (ei(i, b), 0, 0)),
            pl.BlockSpec((1, 1, D), lambda i, b: (ei(i, b), 0, 0)),
        ],
        out_specs=pl.BlockSpec((BLK, D), lambda i, b: (i, 0)),
    )
    return pl.pallas_call(
        body, grid_spec=grid_spec,
        out_shape=jax.ShapeDtypeStruct((PMAX, D), F32),
    )(be, x_sorted, eg, eb, W1b, b1, W2b, b2)


def _head_call(tokens, r1, r2, meta, W_out_b, b_out2, W_dm_b, b_dm2):
    nblk = T // BLK

    def body(t_ref, r1_ref, r2_ref, m_ref, wo_ref, bo_ref, wd_ref, bd_ref,
             cls_ref, dm_ref):
        lane = lax.broadcasted_iota(jnp.int32, (BLK, 128), 1)
        m = m_ref[...]
        g1 = jnp.sum(jnp.where(lane == 0, m, 0.0), axis=1, keepdims=True)
        g2 = jnp.sum(jnp.where(lane == 1, m, 0.0), axis=1, keepdims=True)
        out = t_ref[...] + g1 * r1_ref[...] + g2 * r2_ref[...]
        ob = out.astype(BF16)
        cls_ref[...] = jnp.dot(ob, wo_ref[...],
                               preferred_element_type=F32) + bo_ref[...]
        dm = jnp.dot(ob, wd_ref[...],
                     preferred_element_type=F32) + bd_ref[...]
        dm_ref[...] = jnp.where(lane == E, jax.nn.softplus(dm), dm)

    return pl.pallas_call(
        body,
        grid=(nblk,),
        in_specs=[
            pl.BlockSpec((BLK, D), lambda i: (i, 0)),
            pl.BlockSpec((BLK, D), lambda i: (i, 0)),
            pl.BlockSpec((BLK, D), lambda i: (i, 0)),
            pl.BlockSpec((BLK, 128), lambda i: (i, 0)),
            pl.BlockSpec((D, NCLS), lambda i: (0, 0)),
            pl.BlockSpec((1, NCLS), lambda i: (0, 0)),
            pl.BlockSpec((D, 128), lambda i: (0, 0)),
            pl.BlockSpec((1, 128), lambda i: (0, 0)),
        ],
        out_specs=[
            pl.BlockSpec((BLK, NCLS), lambda i: (i, 0)),
            pl.BlockSpec((BLK, 128), lambda i: (i, 0)),
        ],
        out_shape=[
            jax.ShapeDtypeStruct((T, NCLS), F32),
            jax.ShapeDtypeStruct((T, 128), F32),
        ],
    )(tokens, r1, r2, meta, W_out_b, b_out2, W_dm_b, b_dm2)


def kernel(query_keys, query_start_nodes, writer_keys, writer_labels,
           writer_start_nodes, W_key, b_key, class_embed, role_embed,
           start_embed, ln_g, ln_b, home_hash, expert_ln_g, expert_ln_b,
           W1, b1, W2, b2, W_out, b_out, W_delay, b_delay, W_mag, b_mag):
    wk = writer_keys.reshape(-1, KD).astype(F32)
    keys_all = jnp.concatenate([wk, query_keys.astype(F32)], axis=0)
    nodes_all = jnp.concatenate(
        [writer_start_nodes.reshape(-1), query_start_nodes],
        axis=0).astype(jnp.int32)
    labels = writer_labels.reshape(-1).astype(jnp.int32)
    labels_col = jnp.concatenate(
        [labels, jnp.full((T - TW,), 4096, jnp.int32)]).reshape(T, 1)
    home_pad = jnp.zeros((128, 128), F32).at[:, :E].set(home_hash.astype(F32))
    b_key2 = b_key.reshape(1, D).astype(F32)
    ln_g2 = ln_g.reshape(1, D).astype(F32)
    ln_b2 = ln_b.reshape(1, D).astype(F32)
    b_out2 = b_out.reshape(1, NCLS).astype(F32)
    W_dm = jnp.zeros((D, 128), F32)
    W_dm = W_dm.at[:, :E].set(W_delay.astype(F32))
    W_dm = W_dm.at[:, E:E + 1].set(W_mag.astype(F32))
    b_dm2 = jnp.zeros((1, 128), F32)
    b_dm2 = b_dm2.at[0, :E].set(b_delay.astype(F32))
    b_dm2 = b_dm2.at[0, E:E + 1].set(b_mag.astype(F32))

    ext, cls_rows = _sc_gather_embed(nodes_all, labels,
                                     start_embed.astype(F32),
                                     class_embed.astype(F32))
    tokens, xnorm, logits_pad, meta = _encode_call(
        keys_all, ext, cls_rows, labels_col, W_key.astype(F32), b_key2,
        role_embed.astype(F32).reshape(3, 1, D), ln_g2, ln_b2, home_pad)
    d1c, d2c, bec = _route_call(meta)
    dest1 = d1c[:, 0]
    dest2 = d2c[:, 0]
    be = bec[:, 0]
    x_sorted = _sc_scatter_rows(xnorm, dest1, dest2)
    y_sorted = _ffn_call(be, x_sorted,
                         expert_ln_g.astype(F32).reshape(E, 1, D),
                         expert_ln_b.astype(F32).reshape(E, 1, D),
                         W1.astype(BF16),
                         b1.astype(F32).reshape(E, 1, FFD),
                         W2.astype(BF16),
                         b2.astype(F32).reshape(E, 1, D))
    r1, r2 = _sc_gather_rows(y_sorted, dest1, dest2)
    class_logits, dm = _head_call(tokens, r1, r2, meta,
                                  W_out.astype(BF16), b_out2,
                                  W_dm.astype(BF16), b_dm2)
    delay_logits = dm[:, :E]
    magnitude = dm[:, E:E + 1]
    router_logits = logits_pad[:, :E]
    return class_logits, delay_logits, magnitude, router_logits


# speedup vs baseline: 1.3975x; 1.0384x over previous
import functools

import jax
import jax.numpy as jnp
from jax import lax
from jax.experimental import pallas as pl
from jax.experimental.pallas import tpu as pltpu
from jax.experimental.pallas import tpu_sc as plsc

F32 = jnp.float32
BF16 = jnp.bfloat16
T = 3072
TW = 1024
D = 1024
KD = 128
FFD = 2048
E = 8
NCLS = 256
BLK = 256
NB = 32
PMAX = NB * BLK
NWRK = 32
TPW = T // NWRK
CH = 48
CPW = TW // NWRK


def _sc_gather_embed(nodes_all, labels, start_embed, class_embed):
    mesh = plsc.VectorSubcoreMesh(core_axis_name="c", subcore_axis_name="s")

    @functools.partial(
        pl.kernel, mesh=mesh,
        out_type=(jax.ShapeDtypeStruct((T, D), F32),
                  jax.ShapeDtypeStruct((TW, D), F32)),
        scratch_types=[pltpu.VMEM((CH,), jnp.int32),
                       pltpu.VMEM((CH, D), F32),
                       pltpu.VMEM((CPW,), jnp.int32),
                       pltpu.VMEM((CPW, D), F32),
                       pltpu.SemaphoreType.DMA],
    )
    def k(nodes_h, labels_h, start_h, class_h, ext_h, cls_h,
          idx_v, rows_v, lidx_v, crows_v, sem):
        wid = lax.axis_index("s") * 2 + lax.axis_index("c")
        base = wid * TPW
        for h in range(TPW // CH):
            off = base + h * CH
            pltpu.sync_copy(nodes_h.at[pl.ds(off, CH)], idx_v)
            pltpu.async_copy(start_h.at[idx_v], rows_v, sem).wait()
            pltpu.sync_copy(rows_v, ext_h.at[pl.ds(off, CH)])
        cb = wid * CPW
        pltpu.sync_copy(labels_h.at[pl.ds(cb, CPW)], lidx_v)
        pltpu.async_copy(class_h.at[lidx_v], crows_v, sem).wait()
        pltpu.sync_copy(crows_v, cls_h.at[pl.ds(cb, CPW)])

    return k(nodes_all, labels, start_embed, class_embed)


def _sc_scatter_rows(xnorm, dest1, dest2):
    mesh = plsc.VectorSubcoreMesh(core_axis_name="c", subcore_axis_name="s")

    @functools.partial(
        pl.kernel, mesh=mesh,
        out_type=jax.ShapeDtypeStruct((PMAX, D), F32),
        scratch_types=[pltpu.VMEM((CH,), jnp.int32),
                       pltpu.VMEM((CH, D), F32),
                       pltpu.SemaphoreType.DMA],
    )
    def k(xn_h, d1_h, d2_h, xs_h, idx_v, rows_v, sem):
        wid = lax.axis_index("s") * 2 + lax.axis_index("c")
        base = wid * TPW
        for h in range(TPW // CH):
            off = base + h * CH
            pltpu.sync_copy(xn_h.at[pl.ds(off, CH)], rows_v)
            pltpu.sync_copy(d1_h.at[pl.ds(off, CH)], idx_v)
            pltpu.async_copy(rows_v, xs_h.at[idx_v], sem).wait()
            pltpu.sync_copy(d2_h.at[pl.ds(off, CH)], idx_v)
            pltpu.async_copy(rows_v, xs_h.at[idx_v], sem).wait()

    return k(xnorm, dest1, dest2)


def _sc_gather_rows(y_sorted, dest1, dest2):
    mesh = plsc.VectorSubcoreMesh(core_axis_name="c", subcore_axis_name="s")

    @functools.partial(
        pl.kernel, mesh=mesh,
        out_type=(jax.ShapeDtypeStruct((T, D), F32),
                  jax.ShapeDtypeStruct((T, D), F32)),
        scratch_types=[pltpu.VMEM((CH,), jnp.int32),
                       pltpu.VMEM((CH, D), F32),
                       pltpu.SemaphoreType.DMA],
    )
    def k(ys_h, d1_h, d2_h, r1_h, r2_h, idx_v, rows_v, sem):
        wid = lax.axis_index("s") * 2 + lax.axis_index("c")
        base = wid * TPW
        for h in range(TPW // CH):
            off = base + h * CH
            pltpu.sync_copy(d1_h.at[pl.ds(off, CH)], idx_v)
            pltpu.async_copy(ys_h.at[idx_v], rows_v, sem).wait()
            pltpu.sync_copy(rows_v, r1_h.at[pl.ds(off, CH)])
            pltpu.sync_copy(d2_h.at[pl.ds(off, CH)], idx_v)
            pltpu.async_copy(ys_h.at[idx_v], rows_v, sem).wait()
            pltpu.sync_copy(rows_v, r2_h.at[pl.ds(off, CH)])

    return k(y_sorted, dest1, dest2)


def _encode_call(keys_all, ext, cls_rows, labels_col, W_key, b_key2,
                 role_embed, ln_g2, ln_b2):
    nblk = T // BLK
    wblk = TW // BLK

    def body(keys_ref, ext_ref, cls_ref, lab_ref, wk_ref, bk_ref, role_ref,
             g_ref, b_ref, tok_ref, xn_ref):
        i = pl.program_id(0)
        k = keys_ref[...]
        learned = jnp.dot(k, wk_ref[...], precision=lax.Precision.HIGHEST,
                          preferred_element_type=F32)
        learned = learned + bk_ref[...] + ext_ref[...] + role_ref[0]
        wmask = jnp.where(i < wblk, 1.0, 0.0).astype(F32)
        learned = learned + wmask * cls_ref[...]
        mu = jnp.mean(learned, axis=-1, keepdims=True)
        var = jnp.mean((learned - mu) ** 2, axis=-1, keepdims=True)
        ln = (learned - mu) / jnp.sqrt(var + 1e-5) * g_ref[...] + b_ref[...]
        col = lax.broadcasted_iota(jnp.int32, (BLK, D), 1)
        lab = lab_ref[...]
        oh = jnp.where(col == lab + KD, 1.0, 0.0).astype(F32)
        kpad = jnp.concatenate([k, jnp.zeros((BLK, D - KD), F32)], axis=1)
        tok = ln + kpad + oh
        tok_ref[...] = tok.astype(BF16)
        mu2 = jnp.mean(tok, axis=-1, keepdims=True)
        var2 = jnp.mean((tok - mu2) ** 2, axis=-1, keepdims=True)
        xn_ref[...] = (tok - mu2) / jnp.sqrt(var2 + 1e-5)

    return pl.pallas_call(
        body,
        grid=(nblk,),
        in_specs=[
            pl.BlockSpec((BLK, KD), lambda i: (i, 0)),
            pl.BlockSpec((BLK, D), lambda i: (i, 0)),
            pl.BlockSpec((BLK, D), lambda i: (jnp.minimum(i, wblk - 1), 0)),
            pl.BlockSpec((BLK, 1), lambda i: (i, 0)),
            pl.BlockSpec((KD, D), lambda i: (0, 0)),
            pl.BlockSpec((1, D), lambda i: (0, 0)),
            pl.BlockSpec((1, 1, D), lambda i: (jnp.where(i < wblk, 0, 1), 0, 0)),
            pl.BlockSpec((1, D), lambda i: (0, 0)),
            pl.BlockSpec((1, D), lambda i: (0, 0)),
        ],
        out_specs=[
            pl.BlockSpec((BLK, D), lambda i: (i, 0)),
            pl.BlockSpec((BLK, D), lambda i: (i, 0)),
        ],
        out_shape=[
            jax.ShapeDtypeStruct((T, D), BF16),
            jax.ShapeDtypeStruct((T, D), F32),
        ],
    )(keys_all, ext, cls_rows, labels_col, W_key, b_key2, role_embed,
      ln_g2, ln_b2)


def _route_call(keys_all, home_pad):
    nchunk = T // BLK

    def body(keys_ref, hh_ref, lg_ref, d1_ref, d2_ref, be_ref, g1_ref, g2_ref,
             e1_scr, e2_scr, r1_scr, r2_scr):
        lane = lax.broadcasted_iota(jnp.int32, (BLK, 128), 1)
        lanef = lane.astype(F32)
        row = lax.broadcasted_iota(jnp.int32, (BLK, BLK), 0)
        col = lax.broadcasted_iota(jnp.int32, (BLK, BLK), 1)
        tri = jnp.where(col < row, 1.0, 0.0).astype(F32)
        neg = jnp.float32(-1e30)

        def pass1(i, carry):
            sl = pl.ds(i * BLK, BLK)
            k = keys_ref[sl, :]
            lg = jnp.dot(k, hh_ref[...], precision=lax.Precision.HIGHEST,
                         preferred_element_type=F32)
            lg_ref[sl, :] = lg
            masked = jnp.where(lane < E, lg, neg)
            m1 = jnp.max(masked, axis=-1, keepdims=True)
            i1 = jnp.min(jnp.where(masked == m1, lanef, jnp.float32(1e9)),
                         axis=-1, keepdims=True)
            masked2 = jnp.where(lanef == i1, neg, masked)
            m2 = jnp.max(masked2, axis=-1, keepdims=True)
            i2 = jnp.min(jnp.where(masked2 == m2, lanef, jnp.float32(1e9)),
                         axis=-1, keepdims=True)
            g1 = 1.0 / (1.0 + jnp.exp(m2 - m1))
            g1_ref[sl, :] = g1
            g2_ref[sl, :] = 1.0 - g1
            e1_scr[sl, :] = i1
            e2_scr[sl, :] = i2
            a1 = jnp.where((lanef == i1) & (lane < E), 1.0, 0.0)
            ex1 = jnp.dot(tri, a1, precision=lax.Precision.HIGHEST,
                          preferred_element_type=F32) + carry
            r1_scr[sl, :] = jnp.sum(a1 * ex1, axis=1, keepdims=True)
            carry = carry + jnp.sum(a1, axis=0, keepdims=True)
            a2 = jnp.where((lanef == i2) & (lane < E), 1.0, 0.0)
            ex2 = jnp.dot(tri, a2, precision=lax.Precision.HIGHEST,
                          preferred_element_type=F32) + carry
            r2_scr[sl, :] = jnp.sum(a2 * ex2, axis=1, keepdims=True)
            return carry + jnp.sum(a2, axis=0, keepdims=True)

        counts = lax.fori_loop(0, nchunk, pass1, jnp.zeros((1, 128), F32))

        pc = jnp.ceil(counts / BLK) * BLK
        r8 = lax.broadcasted_iota(jnp.int32, (128, 128), 0)
        c8 = lax.broadcasted_iota(jnp.int32, (128, 128), 1)
        upper = jnp.where(r8 < c8, 1.0, 0.0).astype(F32)
        offs = jnp.dot(pc, upper, precision=lax.Precision.HIGHEST,
                       preferred_element_type=F32)

        def dest_pass(e_scr, r_scr, d_ref):
            def bdy(i, _):
                e = e_scr[pl.ds(i * BLK, BLK), :]
                a = jnp.where((lanef == e) & (lane < E), 1.0, 0.0)
                oa = jnp.sum(a * offs, axis=1, keepdims=True)
                rank = r_scr[pl.ds(i * BLK, BLK), :]
                d_ref[pl.ds(i * BLK, BLK), :] = (oa + rank).astype(jnp.int32)
                return 0
            lax.fori_loop(0, nchunk, bdy, 0)

        dest_pass(e1_scr, r1_scr, d1_ref)
        dest_pass(e2_scr, r2_scr, d2_ref)

        nb = pc / BLK
        cumnb = jnp.dot(nb, upper, precision=lax.Precision.HIGHEST,
                        preferred_element_type=F32)
        tb = jnp.sum(nb)
        lane_b = lax.broadcasted_iota(jnp.int32, (NB, 128), 1)
        jrow = lax.broadcasted_iota(jnp.int32, (NB, 128), 0).astype(F32)
        ge = jnp.where((jrow >= cumnb) & (lane_b < E), 1.0, 0.0)
        be = jnp.sum(ge, axis=1, keepdims=True) - 1.0
        be = jnp.where(jrow[:, :1] < tb, be, -1.0)
        be_ref[...] = be.astype(jnp.int32)

    return pl.pallas_call(
        body,
        out_shape=[
            jax.ShapeDtypeStruct((T, 128), F32),
            jax.ShapeDtypeStruct((T, 1), jnp.int32),
            jax.ShapeDtypeStruct((T, 1), jnp.int32),
            jax.ShapeDtypeStruct((NB, 1), jnp.int32),
            jax.ShapeDtypeStruct((T, 1), F32),
            jax.ShapeDtypeStruct((T, 1), F32),
        ],
        scratch_shapes=[
            pltpu.VMEM((T, 1), F32),
            pltpu.VMEM((T, 1), F32),
            pltpu.VMEM((T, 1), F32),
            pltpu.VMEM((T, 1), F32),
        ],
    )(keys_all, home_pad)


def _ffn_call(be, x_sorted, eg, eb, W1b, b1, W2b, b2):
    def body(be_ref, x_ref, g_ref, bb_ref, w1_ref, b1_ref, w2_ref, b2_ref,
             y_ref):
        i = pl.program_id(0)

        @pl.when(be_ref[i] >= 0)
        def _():
            h = x_ref[...] * g_ref[0] + bb_ref[0]
            a = jnp.dot(h.astype(BF16), w1_ref[0],
                        preferred_element_type=F32) + b1_ref[0]
            act = jax.nn.gelu(a)
            y_ref[...] = jnp.dot(act.astype(BF16), w2_ref[0],
                                 preferred_element_type=F32) + b2_ref[0]

    def ei(i, b):
        return jnp.maximum(b[i], 0)

    grid_spec = pltpu.PrefetchScalarGridSpec(
        num_scalar_prefetch=1,
        grid=(NB,),
        in_specs=[
            pl.BlockSpec((BLK, D), lambda i, b: (i, 0)),
            pl.BlockSpec((1, 1, D), lambda i, b: (ei(i, b), 0, 0)),
            pl.BlockSpec((1, 1, D), lambda i, b: (ei(i, b), 0, 0)),
            pl.BlockSpec((1, D, FFD), lambda i, b: (ei(i, b), 0, 0)),
            pl.BlockSpec((1, 1, FFD), lambda i, b: (ei(i, b), 0, 0)),
            pl.BlockSpec((1, FFD, D), lambda i, b: (ei(i, b), 0, 0)),
            pl.BlockSpec((1, 1, D), lambda i, b: (ei(i, b), 0, 0)),
        ],
        out_specs=pl.BlockSpec((BLK, D), lambda i, b: (i, 0)),
    )
    return pl.pallas_call(
        body, grid_spec=grid_spec,
        out_shape=jax.ShapeDtypeStruct((PMAX, D), F32),
    )(be, x_sorted, eg, eb, W1b, b1, W2b, b2)


def _head_call(tokens, r1, r2, g1c, g2c, W_out_b, b_out2, W_dm_b, b_dm2):
    nblk = T // BLK

    def body(t_ref, r1_ref, r2_ref, g1_ref, g2_ref, wo_ref, bo_ref, wd_ref,
             bd_ref, cls_ref, dm_ref):
        lane = lax.broadcasted_iota(jnp.int32, (BLK, 128), 1)
        out = (t_ref[...].astype(F32)
               + g1_ref[...] * r1_ref[...] + g2_ref[...] * r2_ref[...])
        ob = out.astype(BF16)
        cls_ref[...] = jnp.dot(ob, wo_ref[...],
                               preferred_element_type=F32) + bo_ref[...]
        dm = jnp.dot(ob, wd_ref[...],
                     preferred_element_type=F32) + bd_ref[...]
        dm_ref[...] = jnp.where(lane == E, jax.nn.softplus(dm), dm)

    return pl.pallas_call(
        body,
        grid=(nblk,),
        in_specs=[
            pl.BlockSpec((BLK, D), lambda i: (i, 0)),
            pl.BlockSpec((BLK, D), lambda i: (i, 0)),
            pl.BlockSpec((BLK, D), lambda i: (i, 0)),
            pl.BlockSpec((BLK, 1), lambda i: (i, 0)),
            pl.BlockSpec((BLK, 1), lambda i: (i, 0)),
            pl.BlockSpec((D, NCLS), lambda i: (0, 0)),
            pl.BlockSpec((1, NCLS), lambda i: (0, 0)),
            pl.BlockSpec((D, 128), lambda i: (0, 0)),
            pl.BlockSpec((1, 128), lambda i: (0, 0)),
        ],
        out_specs=[
            pl.BlockSpec((BLK, NCLS), lambda i: (i, 0)),
            pl.BlockSpec((BLK, 128), lambda i: (i, 0)),
        ],
        out_shape=[
            jax.ShapeDtypeStruct((T, NCLS), F32),
            jax.ShapeDtypeStruct((T, 128), F32),
        ],
    )(tokens, r1, r2, g1c, g2c, W_out_b, b_out2, W_dm_b, b_dm2)


def kernel(query_keys, query_start_nodes, writer_keys, writer_labels,
           writer_start_nodes, W_key, b_key, class_embed, role_embed,
           start_embed, ln_g, ln_b, home_hash, expert_ln_g, expert_ln_b,
           W1, b1, W2, b2, W_out, b_out, W_delay, b_delay, W_mag, b_mag):
    wk = writer_keys.reshape(-1, KD).astype(F32)
    keys_all = jnp.concatenate([wk, query_keys.astype(F32)], axis=0)
    nodes_all = jnp.concatenate(
        [writer_start_nodes.reshape(-1), query_start_nodes],
        axis=0).astype(jnp.int32)
    labels = writer_labels.reshape(-1).astype(jnp.int32)
    labels_col = jnp.concatenate(
        [labels, jnp.full((T - TW,), 4096, jnp.int32)]).reshape(T, 1)
    home_pad = jnp.zeros((128, 128), F32).at[:, :E].set(home_hash.astype(F32))
    b_key2 = b_key.reshape(1, D).astype(F32)
    ln_g2 = ln_g.reshape(1, D).astype(F32)
    ln_b2 = ln_b.reshape(1, D).astype(F32)
    b_out2 = b_out.reshape(1, NCLS).astype(F32)
    W_dm = jnp.zeros((D, 128), F32)
    W_dm = W_dm.at[:, :E].set(W_delay.astype(F32))
    W_dm = W_dm.at[:, E:E + 1].set(W_mag.astype(F32))
    b_dm2 = jnp.zeros((1, 128), F32)
    b_dm2 = b_dm2.at[0, :E].set(b_delay.astype(F32))
    b_dm2 = b_dm2.at[0, E:E + 1].set(b_mag.astype(F32))

    ext, cls_rows = _sc_gather_embed(nodes_all, labels,
                                     start_embed.astype(F32),
                                     class_embed.astype(F32))
    logits_pad, d1c, d2c, bec, g1c, g2c = _route_call(keys_all, home_pad)
    tokens, xnorm = _encode_call(
        keys_all, ext, cls_rows, labels_col, W_key.astype(F32), b_key2,
        role_embed.astype(F32).reshape(3, 1, D), ln_g2, ln_b2)
    dest1 = d1c[:, 0]
    dest2 = d2c[:, 0]
    be = bec[:, 0]
    x_sorted = _sc_scatter_rows(xnorm, dest1, dest2)
    y_sorted = _ffn_call(be, x_sorted,
                         expert_ln_g.astype(F32).reshape(E, 1, D),
                         expert_ln_b.astype(F32).reshape(E, 1, D),
                         W1.astype(BF16),
                         b1.astype(F32).reshape(E, 1, FFD),
                         W2.astype(BF16),
                         b2.astype(F32).reshape(E, 1, D))
    r1, r2 = _sc_gather_rows(y_sorted, dest1, dest2)
    class_logits, dm = _head_call(tokens, r1, r2, g1c, g2c,
                                  W_out.astype(BF16), b_out2,
                                  W_dm.astype(BF16), b_dm2)
    delay_logits = dm[:, :E]
    magnitude = dm[:, E:E + 1]
    router_logits = logits_pad[:, :E]
    return class_logits, delay_logits, magnitude, router_logits
